# baseline (speedup 1.0000x reference)
import functools
import math

import jax
import jax.numpy as jnp
from jax import lax
from jax.experimental import pallas as pl
from jax.experimental.pallas import tpu as pltpu
from jax.experimental.pallas import tpu_sc as plsc

_N = 16
_K = 1 << 20
_TEMP = 0.5
_NC = 2
_NS = 16
_LANES = 16
_ROWS_PER_CORE = _N // _NC
_W = _K // _NS
_CH = 8192
_NCHUNK = _W // _CH
_GROUP = 8
_VPG = _CH // (_GROUP * _LANES)

_LN2 = 0.6931471805599453
_SQRT2 = 1.4142135623730951
_LOG_SCALE = math.lgamma(_K) + (_K - 1) * math.log(_TEMP)


def _vlog_many(xs):
    f = jnp.float32
    xis = [plsc.bitcast(x, jnp.int32) for x in xs]
    es = [(xi >> jnp.int32(23)) - jnp.int32(127) for xi in xis]
    mis = [(xi & jnp.int32(0x007FFFFF)) | jnp.int32(0x3F800000) for xi in xis]
    ms = [plsc.bitcast(mi, jnp.float32) for mi in mis]
    bigs = [m > f(_SQRT2) for m in ms]
    ms = [jnp.where(b, m * f(0.5), m) for b, m in zip(bigs, ms)]
    es = [jnp.where(b, e + jnp.int32(1), e) for b, e in zip(bigs, es)]
    efs = [e.astype(jnp.float32) for e in es]
    ss = [(m - f(1.0)) / (m + f(1.0)) for m in ms]
    zs = [s * s for s in ss]
    ps = [z * f(2.0 / 9.0) + f(2.0 / 7.0) for z in zs]
    ps = [p * z + f(2.0 / 5.0) for p, z in zip(ps, zs)]
    ps = [p * z + f(2.0 / 3.0) for p, z in zip(ps, zs)]
    ps = [p * z + f(2.0) for p, z in zip(ps, zs)]
    return [ef * f(_LN2) + s * p for ef, s, p in zip(efs, ss, ps)]


def _tree_sum(vs):
    while len(vs) > 1:
        vs = [a + b for a, b in zip(vs[0::2], vs[1::2])]
    return vs[0]


@functools.partial(
    pl.kernel,
    out_type=(jax.ShapeDtypeStruct((_N, _K), jnp.float32),
              jax.ShapeDtypeStruct((_N, _LANES), jnp.float32)),
    mesh=plsc.VectorSubcoreMesh(core_axis_name="c", subcore_axis_name="s"),
    compiler_params=pltpu.CompilerParams(needs_layout_passes=False),
    scratch_types=[
        pltpu.VMEM((_W,), jnp.float32),
        pltpu.VMEM((2, _CH), jnp.float32),
        pltpu.VMEM((2, _CH), jnp.float32),
        pltpu.VMEM((_LANES,), jnp.float32),
        pltpu.VMEM((_NS * _LANES,), jnp.float32),
        pltpu.VMEM_SHARED((_NS * _LANES,), jnp.float32),
        pltpu.SemaphoreType.DMA((2,)),
        pltpu.SemaphoreType.DMA((2,)),
    ],
)
def _sc_sample(u_hbm, lg_hbm, out_hbm, stats_hbm,
               ebuf, ubuf, lbuf, pbuf, sbuf, shared, usem, lsem):
    c = lax.axis_index("c")
    s = lax.axis_index("s")
    col0 = s * _W
    lane = lax.iota(jnp.int32, 16)
    zero = jnp.zeros((_LANES,), jnp.float32)

    def row_step(r_local, carry):
        r = c * _ROWS_PER_CORE + r_local

        def start_chunk(ci, b):
            off = col0 + ci * _CH
            pltpu.make_async_copy(
                u_hbm.at[r, pl.ds(off, _CH)], ubuf.at[b], usem.at[b]).start()
            pltpu.make_async_copy(
                lg_hbm.at[pl.ds(off, _CH)], lbuf.at[b], lsem.at[b]).start()

        def wait_chunk(ci, b):
            off = col0 + ci * _CH
            pltpu.make_async_copy(
                u_hbm.at[r, pl.ds(off, _CH)], ubuf.at[b], usem.at[b]).wait()
            pltpu.make_async_copy(
                lg_hbm.at[pl.ds(off, _CH)], lbuf.at[b], lsem.at[b]).wait()

        start_chunk(0, 0)

        def chunk_step(ci, accs):
            acc_t, acc_lnt, acc_e, acc_l = accs
            par = lax.rem(ci, 2)

            @pl.when(ci + 1 < _NCHUNK)
            def _():
                start_chunk(ci + 1, 1 - par)

            wait_chunk(ci, par)

            def group_step(gi, gaccs):
                g_t, g_lnt, g_e, g_l = gaccs
                base = gi * (_GROUP * _LANES)
                idxs = [base + k * _LANES for k in range(_GROUP)]
                uvs = [ubuf[par, pl.ds(i, _LANES)] for i in idxs]
                lnus = _vlog_many(uvs)
                ts = [zero - x for x in lnus]
                lvs = [lbuf[par, pl.ds(i, _LANES)] for i in idxs]
                ws = [jnp.exp(lv + lv) for lv in lvs]
                tts = [t * t for t in ts]
                evs = [w / tt for w, tt in zip(ws, tts)]
                for i, ev in zip(idxs, evs):
                    ebuf[pl.ds(ci * _CH + i, _LANES)] = ev
                prods = ts
                while len(prods) > 1:
                    prods = [a * b for a, b in zip(prods[0::2], prods[1::2])]
                g_t = g_t + _tree_sum(ts)
                g_e = g_e + _tree_sum(evs)
                g_l = g_l + _tree_sum(lvs)
                g_lnt = g_lnt + _vlog_many([prods[0]])[0]
                return g_t, g_lnt, g_e, g_l

            return lax.fori_loop(0, _VPG, group_step,
                                 (acc_t, acc_lnt, acc_e, acc_l))

        acc_t, acc_lnt, acc_e, acc_l = lax.fori_loop(
            0, _NCHUNK, chunk_step, (zero, zero, zero, zero))

        s1 = jnp.sum(acc_t)
        s2 = jnp.sum(acc_lnt)
        m = jnp.sum(acc_e)
        lsum = jnp.sum(acc_l)
        pvec = jnp.where(
            lane == 0, s1,
            jnp.where(lane == 1, s2,
                      jnp.where(lane == 2, m,
                                jnp.where(lane == 3, lsum,
                                          jnp.float32(0.0)))))
        pbuf[...] = pvec
        plsc.subcore_barrier()
        pltpu.sync_copy(pbuf, shared.at[pl.ds(s * _LANES, _LANES)])
        plsc.subcore_barrier()
        pltpu.sync_copy(shared, sbuf)

        def red_step(i, tot):
            return tot + sbuf[pl.ds(i * _LANES, _LANES)]

        tot = lax.fori_loop(0, _NS, red_step, zero)

        @pl.when(s == 0)
        def _():
            pbuf[...] = tot
            pltpu.sync_copy(pbuf, stats_hbm.at[r])

        mtot = jnp.sum(jnp.where(lane == 2, tot, jnp.float32(0.0)))
        invv = jnp.full((_LANES,), 1.0, jnp.float32) / jnp.broadcast_to(
            mtot, (_LANES,))

        def scale_step(vi, _):
            base = vi * (_GROUP * _LANES)
            for k in range(_GROUP):
                i = base + k * _LANES
                ebuf[pl.ds(i, _LANES)] = ebuf[pl.ds(i, _LANES)] * invv
            return _

        lax.fori_loop(0, _W // (_GROUP * _LANES), scale_step, 0)
        pltpu.sync_copy(ebuf, out_hbm.at[r, pl.ds(col0, _W)])
        return carry

    lax.fori_loop(0, _ROWS_PER_CORE, row_step, 0)


def kernel(input, logits):
    sample, stats = _sc_sample(input, logits)
    s1 = stats[:, 0]
    s2 = stats[:, 1]
    m = stats[:, 2]
    lsum = stats[:, 3]
    log_prob = (jnp.float32(3.0) * s2
                + jnp.float32(_K) * (jnp.log(m) - jnp.log(s1))
                - jnp.float32(2.0) * lsum
                + jnp.float32(_LOG_SCALE))
    return sample, log_prob


# speedup vs baseline: 1.5237x; 1.5237x over previous
import functools
import math

import jax
import jax.numpy as jnp
from jax import lax
from jax.experimental import pallas as pl
from jax.experimental.pallas import tpu as pltpu
from jax.experimental.pallas import tpu_sc as plsc

_N = 16
_K = 1 << 20
_TEMP = 0.5
_N_SC = 4
_N_TC = _N - _N_SC
_NC = 2
_NS = 16
_LANES = 16
_ROWS_PER_CORE = _N_SC // _NC
_W = _K // _NS
_CH = 8192
_NCHUNK = _W // _CH
_GROUP = 8
_VPG = _CH // (_GROUP * _LANES)

_SUB = 8192

_LN2 = 0.6931471805599453
_SQRT2 = 1.4142135623730951
_LOG_SCALE = math.lgamma(_K) + (_K - 1) * math.log(_TEMP)


def _vlog_many(xs):
    f = jnp.float32
    xis = [plsc.bitcast(x, jnp.int32) for x in xs]
    es = [(xi >> jnp.int32(23)) - jnp.int32(127) for xi in xis]
    mis = [(xi & jnp.int32(0x007FFFFF)) | jnp.int32(0x3F800000) for xi in xis]
    ms = [plsc.bitcast(mi, jnp.float32) for mi in mis]
    bigs = [m > f(_SQRT2) for m in ms]
    ms = [jnp.where(b, m * f(0.5), m) for b, m in zip(bigs, ms)]
    es = [jnp.where(b, e + jnp.int32(1), e) for b, e in zip(bigs, es)]
    efs = [e.astype(jnp.float32) for e in es]
    ss = [(m - f(1.0)) / (m + f(1.0)) for m in ms]
    zs = [s * s for s in ss]
    ps = [z * f(2.0 / 9.0) + f(2.0 / 7.0) for z in zs]
    ps = [p * z + f(2.0 / 5.0) for p, z in zip(ps, zs)]
    ps = [p * z + f(2.0 / 3.0) for p, z in zip(ps, zs)]
    ps = [p * z + f(2.0) for p, z in zip(ps, zs)]
    return [ef * f(_LN2) + s * p for ef, s, p in zip(efs, ss, ps)]


def _tree_sum(vs):
    while len(vs) > 1:
        vs = [a + b for a, b in zip(vs[0::2], vs[1::2])]
    return vs[0]


@functools.partial(
    pl.kernel,
    out_type=(jax.ShapeDtypeStruct((_N_SC, _K), jnp.float32),
              jax.ShapeDtypeStruct((_N_SC, _LANES), jnp.float32)),
    mesh=plsc.VectorSubcoreMesh(core_axis_name="c", subcore_axis_name="s"),
    compiler_params=pltpu.CompilerParams(needs_layout_passes=False),
    scratch_types=[
        pltpu.VMEM((_W,), jnp.float32),
        pltpu.VMEM((2, _CH), jnp.float32),
        pltpu.VMEM((2, _CH), jnp.float32),
        pltpu.VMEM((_LANES,), jnp.float32),
        pltpu.VMEM((_NS * _LANES,), jnp.float32),
        pltpu.VMEM_SHARED((_NS * _LANES,), jnp.float32),
        pltpu.SemaphoreType.DMA((2,)),
        pltpu.SemaphoreType.DMA((2,)),
        pltpu.SemaphoreType.DMA,
    ],
)
def _sc_sample(u_hbm, lg_hbm, out_hbm, stats_hbm,
               ebuf, ubuf, lbuf, pbuf, sbuf, shared, usem, lsem, osem):
    c = lax.axis_index("c")
    s = lax.axis_index("s")
    col0 = s * _W
    lane = lax.iota(jnp.int32, 16)
    zero = jnp.zeros((_LANES,), jnp.float32)

    def row_step(r_local, carry):
        r = c * _ROWS_PER_CORE + r_local
        r_in = _N_TC + r

        def start_chunk(ci, b):
            off = col0 + ci * _CH
            pltpu.make_async_copy(
                u_hbm.at[r_in, pl.ds(off, _CH)], ubuf.at[b], usem.at[b]).start()
            pltpu.make_async_copy(
                lg_hbm.at[pl.ds(off, _CH)], lbuf.at[b], lsem.at[b]).start()

        def wait_chunk(ci, b):
            off = col0 + ci * _CH
            pltpu.make_async_copy(
                u_hbm.at[r_in, pl.ds(off, _CH)], ubuf.at[b], usem.at[b]).wait()
            pltpu.make_async_copy(
                lg_hbm.at[pl.ds(off, _CH)], lbuf.at[b], lsem.at[b]).wait()

        start_chunk(0, 0)

        def chunk_step(ci, accs):
            acc_t, acc_lnt, acc_e, acc_l = accs
            par = lax.rem(ci, 2)

            @pl.when(ci + 1 < _NCHUNK)
            def _():
                start_chunk(ci + 1, 1 - par)

            wait_chunk(ci, par)

            def group_step(gi, gaccs):
                g_t, g_lnt, g_e, g_l = gaccs
                base = gi * (_GROUP * _LANES)
                idxs = [base + k * _LANES for k in range(_GROUP)]
                uvs = [ubuf[par, pl.ds(i, _LANES)] for i in idxs]
                lnus = _vlog_many(uvs)
                ts = [zero - x for x in lnus]
                lvs = [lbuf[par, pl.ds(i, _LANES)] for i in idxs]
                ws = [jnp.exp(lv + lv) for lv in lvs]
                tts = [t * t for t in ts]
                evs = [w / tt for w, tt in zip(ws, tts)]
                for i, ev in zip(idxs, evs):
                    ebuf[pl.ds(ci * _CH + i, _LANES)] = ev
                prods = ts
                while len(prods) > 1:
                    prods = [a * b for a, b in zip(prods[0::2], prods[1::2])]
                g_t = g_t + _tree_sum(ts)
                g_e = g_e + _tree_sum(evs)
                g_l = g_l + _tree_sum(lvs)
                g_lnt = g_lnt + _vlog_many([prods[0]])[0]
                return g_t, g_lnt, g_e, g_l

            return plsc.parallel_loop(
                0, _VPG, unroll=2,
                carry=(acc_t, acc_lnt, acc_e, acc_l))(group_step)

        acc_t, acc_lnt, acc_e, acc_l = lax.fori_loop(
            0, _NCHUNK, chunk_step, (zero, zero, zero, zero))

        s1 = jnp.sum(acc_t)
        s2 = jnp.sum(acc_lnt)
        m = jnp.sum(acc_e)
        lsum = jnp.sum(acc_l)
        pvec = jnp.where(
            lane == 0, s1,
            jnp.where(lane == 1, s2,
                      jnp.where(lane == 2, m,
                                jnp.where(lane == 3, lsum,
                                          jnp.float32(0.0)))))
        pbuf[...] = pvec
        plsc.subcore_barrier()
        pltpu.sync_copy(pbuf, shared.at[pl.ds(s * _LANES, _LANES)])
        plsc.subcore_barrier()
        pltpu.sync_copy(shared, sbuf)

        def red_step(i, tot):
            return tot + sbuf[pl.ds(i * _LANES, _LANES)]

        tot = lax.fori_loop(0, _NS, red_step, zero)

        @pl.when(s == 0)
        def _():
            pbuf[...] = tot
            pltpu.sync_copy(pbuf, stats_hbm.at[r])

        mtot = jnp.sum(jnp.where(lane == 2, tot, jnp.float32(0.0)))
        invv = jnp.full((_LANES,), 1.0, jnp.float32) / jnp.broadcast_to(
            mtot, (_LANES,))

        def out_copy(ci):
            return pltpu.make_async_copy(
                ebuf.at[pl.ds(ci * _CH, _CH)],
                out_hbm.at[r, pl.ds(col0 + ci * _CH, _CH)], osem)

        def scale_chunk(ci, _):
            def scale_step(vi, _s):
                base = ci * _CH + vi * (_GROUP * _LANES)
                for k in range(_GROUP):
                    i = base + k * _LANES
                    ebuf[pl.ds(i, _LANES)] = ebuf[pl.ds(i, _LANES)] * invv
                return _s

            lax.fori_loop(0, _CH // (_GROUP * _LANES), scale_step, 0)
            out_copy(ci).start()
            return _

        lax.fori_loop(0, _NCHUNK, scale_chunk, 0)

        def drain_step(ci, _):
            out_copy(ci).wait()
            return _

        lax.fori_loop(0, _NCHUNK, drain_step, 0)
        return carry

    lax.fori_loop(0, _ROWS_PER_CORE, row_step, 0)


def _tc_body(u_ref, lg_ref, out_ref, st_ref):
    u = u_ref[...]
    lg = lg_ref[...]
    t = -jnp.log(u)
    lnt = jnp.log(t)
    e = jnp.exp((lg - lnt) * jnp.float32(2.0))
    m = jnp.sum(e)
    out_ref[...] = e * (jnp.float32(1.0) / m)
    s1 = jnp.sum(t)
    s2 = jnp.sum(lnt)
    lsum = jnp.sum(lg)
    lane = lax.broadcasted_iota(jnp.int32, (1, 1, 128), 2)
    st_ref[...] = jnp.where(
        lane == 0, s1,
        jnp.where(lane == 1, s2,
                  jnp.where(lane == 2, m,
                            jnp.where(lane == 3, lsum, jnp.float32(0.0)))))


_tc_rows = pl.pallas_call(
    _tc_body,
    grid=(_N_TC,),
    in_specs=[
        pl.BlockSpec((_SUB, 128), lambda i: (i, 0)),
        pl.BlockSpec((_SUB, 128), lambda i: (0, 0)),
    ],
    out_specs=[
        pl.BlockSpec((_SUB, 128), lambda i: (i, 0)),
        pl.BlockSpec((1, 1, 128), lambda i: (i, 0, 0)),
    ],
    out_shape=[
        jax.ShapeDtypeStruct((_N * _SUB, 128), jnp.float32),
        jax.ShapeDtypeStruct((_N_TC, 1, 128), jnp.float32),
    ],
    compiler_params=pltpu.CompilerParams(
        vmem_limit_bytes=100 * 1024 * 1024),
)


def kernel(input, logits):
    sc_sample, sc_stats = _sc_sample(input, logits)
    u2 = input.reshape(_N * _SUB, 128)
    lg2 = logits.reshape(_SUB, 128)
    tc_out, tc_stats = _tc_rows(u2, lg2)
    sample = lax.dynamic_update_slice(
        tc_out.reshape(_N, _K), sc_sample, (_N_TC, 0))
    stats = jnp.concatenate([tc_stats[:, 0, :4], sc_stats[:, :4]], axis=0)
    s1 = stats[:, 0]
    s2 = stats[:, 1]
    m = stats[:, 2]
    lsum = stats[:, 3]
    log_prob = (jnp.float32(3.0) * s2
                + jnp.float32(_K) * (jnp.log(m) - jnp.log(s1))
                - jnp.float32(2.0) * lsum
                + jnp.float32(_LOG_SCALE))
    return sample, log_prob
